# baseline (speedup 1.0000x reference)
import functools

import jax
import jax.numpy as jnp
from jax.experimental import pallas as pl

N_CTX = 2048
N_EMBD = 768
N_EXP = 64
B = 4
FLAT = N_CTX * N_EMBD

CHUNK = 65536
N_STEPS = FLAT // CHUNK


def _router_kernel(x_ref, w_ref, b_ref, o_ref):
    i = pl.program_id(0)
    part = jax.lax.dot_general(
        x_ref[...], w_ref[...],
        dimension_numbers=(((1,), (1,)), ((), ())),
        preferred_element_type=jnp.float32,
    )

    @pl.when(i == 0)
    def _init():
        o_ref[...] = part

    @pl.when(i > 0)
    def _acc():
        o_ref[...] = o_ref[...] + part

    @pl.when(i == N_STEPS - 1)
    def _epilogue():
        y = o_ref[...] + b_ref[...]
        col = jax.lax.broadcasted_iota(jnp.int32, (B, N_EXP), 1)
        v1 = jnp.max(y, axis=1, keepdims=True)
        i1 = jnp.min(jnp.where(y == v1, col, N_EXP), axis=1, keepdims=True)
        sel1 = col == i1
        y2 = jnp.where(sel1, -jnp.inf, y)
        v2 = jnp.max(y2, axis=1, keepdims=True)
        i2 = jnp.min(jnp.where(y2 == v2, col, N_EXP), axis=1, keepdims=True)
        sel2 = col == i2
        mask = jnp.where(sel1 | sel2, y, 0.0)
        m = jnp.max(mask, axis=1, keepdims=True)
        e = jnp.exp(mask - m)
        o_ref[...] = e / jnp.sum(e, axis=1, keepdims=True)


@jax.jit
def kernel(x, W, b):
    xf = x.reshape(B, FLAT)
    b2 = b.reshape(1, N_EXP)
    grid = (N_STEPS,)
    return pl.pallas_call(
        _router_kernel,
        grid=grid,
        in_specs=[
            pl.BlockSpec((B, CHUNK), lambda i: (0, i)),
            pl.BlockSpec((N_EXP, CHUNK), lambda i: (0, i)),
            pl.BlockSpec((1, N_EXP), lambda i: (0, 0)),
        ],
        out_specs=pl.BlockSpec((B, N_EXP), lambda i: (0, 0)),
        out_shape=jax.ShapeDtypeStruct((B, N_EXP), jnp.float32),
    )(xf, W, b2)


# speedup vs baseline: 1.0146x; 1.0146x over previous
import functools

import jax
import jax.numpy as jnp
from jax.experimental import pallas as pl

N_CTX = 2048
N_EMBD = 768
N_EXP = 64
B = 4
FLAT = N_CTX * N_EMBD

CHUNK = 32768
N_STEPS = FLAT // CHUNK


def _router_kernel(x_ref, w_ref, b_ref, o_ref):
    i = pl.program_id(0)
    part = jax.lax.dot_general(
        x_ref[...], w_ref[...],
        dimension_numbers=(((1,), (1,)), ((), ())),
        preferred_element_type=jnp.float32,
    )

    @pl.when(i == 0)
    def _init():
        o_ref[...] = part

    @pl.when(i > 0)
    def _acc():
        o_ref[...] = o_ref[...] + part

    @pl.when(i == N_STEPS - 1)
    def _epilogue():
        y = o_ref[...] + b_ref[...]
        col = jax.lax.broadcasted_iota(jnp.int32, (B, N_EXP), 1)
        v1 = jnp.max(y, axis=1, keepdims=True)
        i1 = jnp.min(jnp.where(y == v1, col, N_EXP), axis=1, keepdims=True)
        sel1 = col == i1
        y2 = jnp.where(sel1, -jnp.inf, y)
        v2 = jnp.max(y2, axis=1, keepdims=True)
        i2 = jnp.min(jnp.where(y2 == v2, col, N_EXP), axis=1, keepdims=True)
        sel2 = col == i2
        mask = jnp.where(sel1 | sel2, y, 0.0)
        m = jnp.max(mask, axis=1, keepdims=True)
        e = jnp.exp(mask - m)
        o_ref[...] = e / jnp.sum(e, axis=1, keepdims=True)


@jax.jit
def kernel(x, W, b):
    xf = x.reshape(B, FLAT)
    b2 = b.reshape(1, N_EXP)
    grid = (N_STEPS,)
    return pl.pallas_call(
        _router_kernel,
        grid=grid,
        in_specs=[
            pl.BlockSpec((B, CHUNK), lambda i: (0, i)),
            pl.BlockSpec((N_EXP, CHUNK), lambda i: (0, i)),
            pl.BlockSpec((1, N_EXP), lambda i: (0, 0)),
        ],
        out_specs=pl.BlockSpec((B, N_EXP), lambda i: (0, 0)),
        out_shape=jax.ShapeDtypeStruct((B, N_EXP), jnp.float32),
    )(xf, W, b2)
